# baseline (speedup 1.0000x reference)
import functools

import jax
import jax.numpy as jnp
import numpy as np
from jax.experimental import pallas as pl
from jax.experimental.pallas import tpu as pltpu

_DX = np.float32(0.3)
_DZ = np.float32(20.0)
_OX = np.float32(np.float32(-54.0 + 0.3 / 2.0) - _DX / np.float32(2.0))
_OY = _OX
_OZ = np.float32(np.float32(-10.0 + 20.0 / 2.0) - _DZ / np.float32(2.0))
_NXY = 360
_NVOX = _NXY * _NXY
_NROWS = _NVOX + 64
_D, _FH, _FW = 30, 32, 88
_NCAM = 6
_PTS_PER_CAM = _D * _FH * _FW
_NPTS = _NCAM * _PTS_PER_CAM
_C = 80


def _index_kernel(gx_ref, gy_ref, gz_ref, out_ref):
    gx = gx_ref[...]
    gy = gy_ref[...]
    gz = gz_ref[...]
    cxf = jnp.trunc((gx - _OX) / _DX)
    cyf = jnp.trunc((gy - _OY) / _DX)
    czf = jnp.trunc((gz - _OZ) / _DZ)
    cxi = cxf.astype(jnp.int32)
    cyi = cyf.astype(jnp.int32)
    kept = ((cxf >= 0) & (cxf < _NXY) & (cyf >= 0) & (cyf < _NXY)
            & (czf >= 0) & (czf < 1))
    lane = jax.lax.broadcasted_iota(jnp.int32, gx.shape, 1)
    trash = _NVOX + (lane % 64)
    lin = jnp.where(kept, cxi * _NXY + cyi, trash)
    out_ref[...] = lin


def _compute_indices(img_aug_matrix, camera_intrinsics, camera2lidar,
                     lidar_aug_matrix, frustum):
    B, N = 1, _NCAM
    post_rots = img_aug_matrix[..., :3, :3]
    post_trans = img_aug_matrix[..., :3, 3]
    intrins = camera_intrinsics[..., :3, :3]
    c2l_rots = camera2lidar[..., :3, :3]
    c2l_trans = camera2lidar[..., :3, 3]
    extra_rots = lidar_aug_matrix[..., :3, :3]
    extra_trans = lidar_aug_matrix[..., :3, 3]
    pts = frustum - post_trans.reshape(B, N, 1, 1, 1, 3)
    pts = jnp.matmul(jnp.linalg.inv(post_rots).reshape(B, N, 1, 1, 1, 3, 3),
                     pts[..., None])
    pts = jnp.concatenate([pts[..., :2, :] * pts[..., 2:3, :],
                           pts[..., 2:3, :]], axis=-2)
    combine = jnp.matmul(c2l_rots, jnp.linalg.inv(intrins))
    pts = jnp.matmul(combine.reshape(B, N, 1, 1, 1, 3, 3), pts)[..., 0]
    pts = pts + c2l_trans.reshape(B, N, 1, 1, 1, 3)
    pts = jnp.matmul(extra_rots.reshape(B, 1, 1, 1, 1, 3, 3),
                     pts[..., None])[..., 0]
    geom = pts + extra_trans.reshape(B, 1, 1, 1, 1, 3)
    g = geom.reshape(_NPTS, 3)
    gx = g[:, 0].reshape(3960, 128)
    gy = g[:, 1].reshape(3960, 128)
    gz = g[:, 2].reshape(3960, 128)
    lin = pl.pallas_call(
        _index_kernel,
        grid=(5,),
        in_specs=[pl.BlockSpec((792, 128), lambda n: (n, 0))] * 3,
        out_specs=pl.BlockSpec((792, 128), lambda n: (n, 0)),
        out_shape=jax.ShapeDtypeStruct((3960, 128), jnp.int32),
    )(gx, gy, gz)
    return lin.reshape(_NPTS)


def _mean_kernel(sums_ref, cnts_ref, out_ref):
    cnt = cnts_ref[0:1, :] + cnts_ref[1:2, :]
    den = jnp.where(cnt > 0.0, cnt, 1.0)
    for k in range(8):
        t = jnp.swapaxes(sums_ref[k], 0, 1)
        out_ref[k * 10:(k + 1) * 10, :] = t / den


def _mean_transpose(sums8, cnts2):
    VB = 128
    out = pl.pallas_call(
        _mean_kernel,
        grid=(_NROWS // VB,),
        in_specs=[
            pl.BlockSpec((8, VB, 10), lambda v: (0, v, 0)),
            pl.BlockSpec((2, VB), lambda v: (0, v)),
        ],
        out_specs=pl.BlockSpec((_C, VB), lambda v: (0, v)),
        out_shape=jax.ShapeDtypeStruct((_C, _NROWS), jnp.float32),
    )(sums8, cnts2)
    return out[:, :_NVOX].reshape(1, _C, _NXY, _NXY)


def kernel(img, points, lidar2camera, lidar2image, camera_intrinsics,
           camera2lidar, img_aug_matrix, lidar_aug_matrix, metas, frustum):
    lin = _compute_indices(img_aug_matrix, camera_intrinsics, camera2lidar,
                           lidar_aug_matrix, frustum)
    x = img.reshape(_NPTS, _C)
    lin8 = (lin[:, None] + jnp.arange(8, dtype=jnp.int32)[None, :] * _NROWS)
    upd = x.reshape(_NPTS, 8, 10).reshape(_NPTS * 8, 10)
    sums = jnp.zeros((8 * _NROWS, 10), jnp.float32).at[lin8.reshape(-1)].add(upd)
    cnts = jnp.zeros((_NROWS,), jnp.float32).at[lin].add(1.0)
    sums8 = sums.reshape(8, _NROWS, 10)
    cnts2 = jnp.stack([cnts, jnp.zeros((_NROWS,), jnp.float32)])
    return _mean_transpose(sums8, cnts2)


# speedup vs baseline: 3.6355x; 3.6355x over previous
import functools

import jax
import jax.numpy as jnp
import numpy as np
from jax import lax
from jax.experimental import pallas as pl
from jax.experimental.pallas import tpu as pltpu
from jax.experimental.pallas import tpu_sc as plsc

_DX = np.float32(0.3)
_DZ = np.float32(20.0)
_OX = np.float32(np.float32(-54.0 + 0.3 / 2.0) - _DX / np.float32(2.0))
_OY = _OX
_OZ = np.float32(np.float32(-10.0 + 20.0 / 2.0) - _DZ / np.float32(2.0))
_NXY = 360
_NVOX = _NXY * _NXY
_NROWS = 131072
_D, _FH, _FW = 30, 32, 88
_NCAM = 6
_PTS_PER_CAM = _D * _FH * _FW
_NPTS = _NCAM * _PTS_PER_CAM
_C = 80

_NTILES = 16
_ROWS_PER_TILE = _NROWS // _NTILES
_PTS_PER_TILE = _NPTS // _NTILES
_SUB = 2112
_NSUB = _PTS_PER_TILE // _SUB
_CC = 8
_NCHUNK = _C // _CC


def _index_kernel(gx_ref, gy_ref, gz_ref, out_ref):
    gx = gx_ref[...]
    gy = gy_ref[...]
    gz = gz_ref[...]
    cxf = jnp.trunc((gx - _OX) / _DX)
    cyf = jnp.trunc((gy - _OY) / _DX)
    czf = jnp.trunc((gz - _OZ) / _DZ)
    cxi = cxf.astype(jnp.int32)
    cyi = cyf.astype(jnp.int32)
    kept = ((cxf >= 0) & (cxf < _NXY) & (cyf >= 0) & (cyf < _NXY)
            & (czf >= 0) & (czf < 1))
    blk = pl.program_id(0) * (792 * 128)
    flat = (blk + jax.lax.broadcasted_iota(jnp.int32, gx.shape, 0) * 128
            + jax.lax.broadcasted_iota(jnp.int32, gx.shape, 1))
    trash = _NVOX + (flat % 1408)
    lin = jnp.where(kept, cxi * _NXY + cyi, trash)
    out_ref[...] = lin


def _compute_indices(img_aug_matrix, camera_intrinsics, camera2lidar,
                     lidar_aug_matrix, frustum):
    B, N = 1, _NCAM
    post_rots = img_aug_matrix[..., :3, :3]
    post_trans = img_aug_matrix[..., :3, 3]
    intrins = camera_intrinsics[..., :3, :3]
    c2l_rots = camera2lidar[..., :3, :3]
    c2l_trans = camera2lidar[..., :3, 3]
    extra_rots = lidar_aug_matrix[..., :3, :3]
    extra_trans = lidar_aug_matrix[..., :3, 3]
    pts = frustum - post_trans.reshape(B, N, 1, 1, 1, 3)
    pts = jnp.matmul(jnp.linalg.inv(post_rots).reshape(B, N, 1, 1, 1, 3, 3),
                     pts[..., None])
    pts = jnp.concatenate([pts[..., :2, :] * pts[..., 2:3, :],
                           pts[..., 2:3, :]], axis=-2)
    combine = jnp.matmul(c2l_rots, jnp.linalg.inv(intrins))
    pts = jnp.matmul(combine.reshape(B, N, 1, 1, 1, 3, 3), pts)[..., 0]
    pts = pts + c2l_trans.reshape(B, N, 1, 1, 1, 3)
    pts = jnp.matmul(extra_rots.reshape(B, 1, 1, 1, 1, 3, 3),
                     pts[..., None])[..., 0]
    geom = pts + extra_trans.reshape(B, 1, 1, 1, 1, 3)
    g = geom.reshape(_NPTS, 3)
    gx = g[:, 0].reshape(3960, 128)
    gy = g[:, 1].reshape(3960, 128)
    gz = g[:, 2].reshape(3960, 128)
    lin = pl.pallas_call(
        _index_kernel,
        grid=(5,),
        in_specs=[pl.BlockSpec((792, 128), lambda n: (n, 0))] * 3,
        out_specs=pl.BlockSpec((792, 128), lambda n: (n, 0)),
        out_shape=jax.ShapeDtypeStruct((3960, 128), jnp.int32),
    )(gx, gy, gz)
    return lin.reshape(_NPTS)


def _scatter_body(lin_hbm, x_hbm, zero_hbm, zero1_hbm, ones_hbm,
                  sums_hbm, cnts_hbm,
                  acc_sh, cnt_sh, idx_v, feat_v, ones_v, zero_v, zero1_v,
                  drain_v):
    c = lax.axis_index("c")
    s = lax.axis_index("s")
    row0 = s * _ROWS_PER_TILE
    pt0 = s * _PTS_PER_TILE

    pltpu.sync_copy(zero_hbm, zero_v)
    pltpu.sync_copy(zero1_hbm, zero1_v)
    pltpu.sync_copy(ones_hbm, ones_v)
    pltpu.sync_copy(zero1_v, cnt_sh.at[pl.ds(row0, _ROWS_PER_TILE)])

    @pl.loop(0, 5)
    def _pass(p):
        chunk = c * 5 + p
        for j in range(8):
            pltpu.sync_copy(zero_v, acc_sh.at[pl.ds(row0 + j * 1024, 1024), :])
        plsc.subcore_barrier()

        @pl.loop(0, _NSUB)
        def _sub(i):
            base = pt0 + i * _SUB
            pltpu.sync_copy(lin_hbm.at[pl.ds(base, _SUB)], idx_v)
            pltpu.sync_copy(x_hbm.at[chunk, pl.ds(base, _SUB), :], feat_v)
            pltpu.sync_copy(feat_v, acc_sh.at[idx_v], add=True)
            do_cnt = (p == 0) & (((i < 8) & (c == 0)) | ((i >= 8) & (c == 1)))

            @pl.when(do_cnt)
            def _():
                pltpu.sync_copy(ones_v, cnt_sh.at[idx_v], add=True)

        plsc.subcore_barrier()
        for j in range(8):
            r = row0 + j * 1024
            pltpu.sync_copy(acc_sh.at[pl.ds(r, 1024), :], drain_v)
            pltpu.sync_copy(drain_v, sums_hbm.at[chunk, pl.ds(r, 1024), :])

        @pl.when(p == 0)
        def _():
            pltpu.sync_copy(cnt_sh.at[pl.ds(row0, _ROWS_PER_TILE)], zero1_v)
            pltpu.sync_copy(zero1_v,
                            cnts_hbm.at[pl.ds(c * _NROWS + row0,
                                              _ROWS_PER_TILE)])


@functools.cache
def _get_scatter_call():
    return functools.partial(
        pl.kernel,
        out_type=[
            jax.ShapeDtypeStruct((_NCHUNK, _NROWS, _CC), jnp.float32),
            jax.ShapeDtypeStruct((2 * _NROWS,), jnp.float32),
        ],
        mesh=plsc.VectorSubcoreMesh(core_axis_name="c", subcore_axis_name="s"),
        compiler_params=pltpu.CompilerParams(use_tc_tiling_on_sc=False),
        scratch_types=[
            pltpu.VMEM_SHARED((_NROWS, _CC), jnp.float32),
            pltpu.VMEM_SHARED((_NROWS,), jnp.float32),
            pltpu.VMEM((_SUB,), jnp.int32),
            pltpu.VMEM((_SUB, _CC), jnp.float32),
            pltpu.VMEM((_SUB,), jnp.float32),
            pltpu.VMEM((1024, _CC), jnp.float32),
            pltpu.VMEM((_ROWS_PER_TILE,), jnp.float32),
            pltpu.VMEM((1024, _CC), jnp.float32),
        ],
    )(_scatter_body)


def _mean_kernel(sums_ref, cnts_ref, out_ref):
    cnt = cnts_ref[0:1, :] + cnts_ref[1:2, :]
    den = jnp.where(cnt > 0.0, cnt, 1.0)
    for k in range(_NCHUNK):
        t = jnp.swapaxes(sums_ref[k], 0, 1)
        out_ref[k * _CC:(k + 1) * _CC, :] = t / den


def _mean_transpose(sums8, cnts2):
    VB = 128
    out = pl.pallas_call(
        _mean_kernel,
        grid=(_NROWS // VB,),
        in_specs=[
            pl.BlockSpec((_NCHUNK, VB, _CC), lambda v: (0, v, 0)),
            pl.BlockSpec((2, VB), lambda v: (0, v)),
        ],
        out_specs=pl.BlockSpec((_C, VB), lambda v: (0, v)),
        out_shape=jax.ShapeDtypeStruct((_C, _NROWS), jnp.float32),
    )(sums8, cnts2)
    return out[:, :_NVOX].reshape(1, _C, _NXY, _NXY)


def kernel(img, points, lidar2camera, lidar2image, camera_intrinsics,
           camera2lidar, img_aug_matrix, lidar_aug_matrix, metas, frustum):
    lin = _compute_indices(img_aug_matrix, camera_intrinsics, camera2lidar,
                           lidar_aug_matrix, frustum)
    x = img.reshape(_NPTS, _NCHUNK, _CC).transpose(1, 0, 2)
    zero, zero1, ones = lax.optimization_barrier(
        (jnp.zeros((1024, _CC), jnp.float32),
         jnp.zeros((_ROWS_PER_TILE,), jnp.float32),
         jnp.ones((_SUB,), jnp.float32)))
    sums8, cntsf = _get_scatter_call()(lin, x, zero, zero1, ones)
    return _mean_transpose(sums8, cntsf.reshape(2, _NROWS))
